# Initial kernel scaffold; baseline (speedup 1.0000x reference)
import jax
import jax.numpy as jnp
from jax.experimental import pallas as pl


def kernel(x, W0, W_tab, V_tab, mlp_ws, mlp_bs, clf_w, clf_b):
    raise NotImplementedError("write your pallas kernel here")


# speedup vs baseline: 1.7476x; 1.7476x over previous
import functools

import jax
import jax.numpy as jnp
from jax import lax
from jax.experimental import pallas as pl
from jax.experimental.pallas import tpu as pltpu
from jax.experimental.pallas import tpu_sc as plsc

B = 4096
F = 26
K = 64
H = 256
N_LOOKUPS = B * F

_NC = 2
_NS = 16
_NW = _NC * _NS
_BPW = N_LOOKUPS // _NW
_CHUNK = 832
_NCHUNK = _BPW // _CHUNK


def _gather_body(x_hbm, vtab_hbm, wtab_hbm, vout_hbm, wout_hbm,
                 idx_v, vbuf0, vbuf1, wbuf, gsem0, gsem1, wsem):
    wid = lax.axis_index("s") * _NC + lax.axis_index("c")
    base = wid * _BPW
    pltpu.sync_copy(x_hbm.at[pl.ds(base, _BPW)], idx_v)
    wcp = pltpu.async_copy(wtab_hbm.at[idx_v], wbuf, wsem)
    bufs = (vbuf0, vbuf1)
    sems = (gsem0, gsem1)
    cps = [None, None]
    cps[0] = pltpu.async_copy(
        vtab_hbm.at[idx_v.at[pl.ds(0, _CHUNK)]], bufs[0], sems[0])
    for c in range(_NCHUNK):
        cur = c % 2
        nxt = (c + 1) % 2
        if c + 1 < _NCHUNK:
            cps[nxt] = pltpu.async_copy(
                vtab_hbm.at[idx_v.at[pl.ds((c + 1) * _CHUNK, _CHUNK)]],
                bufs[nxt], sems[nxt])
        cps[cur].wait()
        pltpu.sync_copy(bufs[cur],
                        vout_hbm.at[pl.ds(base + c * _CHUNK, _CHUNK)])
    wcp.wait()
    pltpu.sync_copy(wbuf, wout_hbm.at[pl.ds(base, _BPW)])


@functools.cache
def _make_gather():
    return pl.kernel(
        _gather_body,
        out_type=(
            jax.ShapeDtypeStruct((N_LOOKUPS, K), jnp.float32),
            jax.ShapeDtypeStruct((N_LOOKUPS,), jnp.float32),
        ),
        mesh=plsc.VectorSubcoreMesh(core_axis_name="c",
                                    subcore_axis_name="s"),
        compiler_params=pltpu.CompilerParams(use_tc_tiling_on_sc=False),
        scratch_types=(
            pltpu.VMEM((_BPW,), jnp.int32),
            pltpu.VMEM((_CHUNK, K), jnp.float32),
            pltpu.VMEM((_CHUNK, K), jnp.float32),
            pltpu.VMEM((_BPW,), jnp.float32),
            pltpu.SemaphoreType.DMA,
            pltpu.SemaphoreType.DMA,
            pltpu.SemaphoreType.DMA,
        ),
    )

_BB = 256


def _mlp_body(v_ref, wv_ref, w1_ref, wr_ref, ball_ref, clfh_ref, scal_ref,
              out_ref):
    v = v_ref[...]
    s2 = v[:, 0:128] * 1.0
    ss2 = v[:, 0:128] * v[:, 0:128]
    for p in range(1, F // 2):
        c = v[:, p * 128:(p + 1) * 128]
        s2 = s2 + c
        ss2 = ss2 + c * c
    s = s2[:, :K] + s2[:, K:]
    ss = ss2[:, :K] + ss2[:, K:]
    fm2 = 0.5 * jnp.sum(s * s - ss, axis=1, keepdims=True)
    fm1 = jnp.sum(wv_ref[...], axis=1, keepdims=True)
    w0 = scal_ref[0]
    clf_w0 = scal_ref[1]
    clf_b = scal_ref[2]
    fm_out = w0 + fm1 + fm2
    h = jnp.maximum(
        jnp.dot(v, w1_ref[...], preferred_element_type=jnp.float32)
        + ball_ref[0, :], 0.0)
    for i in range(25):
        h = jnp.maximum(
            jnp.dot(h, wr_ref[i], preferred_element_type=jnp.float32)
            + ball_ref[i + 1, :], 0.0)
    out_ref[...] = (fm_out * clf_w0
                    + jnp.dot(h, clfh_ref[...],
                              preferred_element_type=jnp.float32)
                    + clf_b)


@functools.partial(jax.jit, static_argnames=())
def _deepfm_tc(v2, wv, w1, wr, ball, clfh, scal):
    grid = (B // _BB,)
    return pl.pallas_call(
        _mlp_body,
        grid=grid,
        in_specs=[
            pl.BlockSpec((_BB, F * K), lambda i: (i, 0)),
            pl.BlockSpec((_BB, F), lambda i: (i, 0)),
            pl.BlockSpec((F * K, H), lambda i: (0, 0)),
            pl.BlockSpec((25, H, H), lambda i: (0, 0, 0)),
            pl.BlockSpec((26, H), lambda i: (0, 0)),
            pl.BlockSpec((H, 1), lambda i: (0, 0)),
            pl.BlockSpec(memory_space=pltpu.SMEM),
        ],
        out_specs=pl.BlockSpec((_BB, 1), lambda i: (i, 0)),
        out_shape=jax.ShapeDtypeStruct((B, 1), jnp.float32),
    )(v2, wv, w1, wr, ball, clfh, scal)


def kernel(x, W0, W_tab, V_tab, mlp_ws, mlp_bs, clf_w, clf_b):
    x_flat = x.reshape(-1).astype(jnp.int32)
    v_flat, w_flat = _make_gather()(x_flat, V_tab, W_tab.reshape(-1))
    v2 = v_flat.reshape(B, F * K)
    wv = w_flat.reshape(B, F)
    wr = jnp.stack(mlp_ws[1:])
    ball = jnp.stack(mlp_bs)
    clfh = clf_w[1:]
    scal = jnp.concatenate([W0.reshape(-1), clf_w[0].reshape(-1),
                            clf_b.reshape(-1)])
    return _deepfm_tc(v2, wv, mlp_ws[0], wr, ball, clfh, scal)


# speedup vs baseline: 1.8273x; 1.0456x over previous
import functools

import jax
import jax.numpy as jnp
from jax import lax
from jax.experimental import pallas as pl
from jax.experimental.pallas import tpu as pltpu
from jax.experimental.pallas import tpu_sc as plsc

B = 4096
F = 26
K = 64
H = 256
N_LOOKUPS = B * F

_NC = 2
_NS = 16
_NW = _NC * _NS
_BPW = N_LOOKUPS // _NW
_CHUNK = 832
_NCHUNK = _BPW // _CHUNK


def _gather_body(x_hbm, vtab_hbm, wtab_hbm, vout_hbm, wout_hbm,
                 idx_v, vbuf0, vbuf1, wbuf, gsem0, gsem1, wsem):
    wid = lax.axis_index("s") * _NC + lax.axis_index("c")
    base = wid * _BPW
    pltpu.sync_copy(x_hbm.at[pl.ds(base, _BPW)], idx_v)
    wcp = pltpu.async_copy(wtab_hbm.at[idx_v], wbuf, wsem)
    bufs = (vbuf0, vbuf1)
    sems = (gsem0, gsem1)
    cps = [None, None]
    cps[0] = pltpu.async_copy(
        vtab_hbm.at[idx_v.at[pl.ds(0, _CHUNK)]], bufs[0], sems[0])
    for c in range(_NCHUNK):
        cur = c % 2
        nxt = (c + 1) % 2
        if c + 1 < _NCHUNK:
            cps[nxt] = pltpu.async_copy(
                vtab_hbm.at[idx_v.at[pl.ds((c + 1) * _CHUNK, _CHUNK)]],
                bufs[nxt], sems[nxt])
        cps[cur].wait()
        pltpu.sync_copy(bufs[cur],
                        vout_hbm.at[pl.ds(base + c * _CHUNK, _CHUNK)])
    wcp.wait()
    pltpu.sync_copy(wbuf, wout_hbm.at[pl.ds(base, _BPW)])


@functools.cache
def _make_gather():
    return pl.kernel(
        _gather_body,
        out_type=(
            jax.ShapeDtypeStruct((N_LOOKUPS, K), jnp.float32),
            jax.ShapeDtypeStruct((N_LOOKUPS,), jnp.float32),
        ),
        mesh=plsc.VectorSubcoreMesh(core_axis_name="c",
                                    subcore_axis_name="s"),
        compiler_params=pltpu.CompilerParams(use_tc_tiling_on_sc=False),
        scratch_types=(
            pltpu.VMEM((_BPW,), jnp.int32),
            pltpu.VMEM((_CHUNK, K), jnp.float32),
            pltpu.VMEM((_CHUNK, K), jnp.float32),
            pltpu.VMEM((_BPW,), jnp.float32),
            pltpu.SemaphoreType.DMA,
            pltpu.SemaphoreType.DMA,
            pltpu.SemaphoreType.DMA,
        ),
    )

_BB = 256


def _mlp_body(*refs):
    v_ref, wv_ref = refs[0], refs[1]
    w_refs = refs[2:2 + F]
    ball_ref, clfh_ref, scal_ref = refs[2 + F], refs[3 + F], refs[4 + F]
    out_ref = refs[5 + F]
    v = v_ref[...]
    s2 = v[:, 0:128] * 1.0
    ss2 = v[:, 0:128] * v[:, 0:128]
    for p in range(1, F // 2):
        c = v[:, p * 128:(p + 1) * 128]
        s2 = s2 + c
        ss2 = ss2 + c * c
    s = s2[:, :K] + s2[:, K:]
    ss = ss2[:, :K] + ss2[:, K:]
    fm2 = 0.5 * jnp.sum(s * s - ss, axis=1, keepdims=True)
    fm1 = jnp.sum(wv_ref[...], axis=1, keepdims=True)
    w0 = scal_ref[0]
    clf_w0 = scal_ref[1]
    clf_b = scal_ref[2]
    fm_out = w0 + fm1 + fm2
    h = v
    for i in range(F):
        h = jnp.maximum(
            jnp.dot(h, w_refs[i][...], preferred_element_type=jnp.float32)
            + ball_ref[i, :], 0.0)
    out_ref[...] = (fm_out * clf_w0
                    + jnp.dot(h, clfh_ref[...],
                              preferred_element_type=jnp.float32)
                    + clf_b)


def _deepfm_tc(v2, wv, ws, ball, clfh, scal):
    grid = (B // _BB,)
    w_specs = [pl.BlockSpec(w.shape, lambda i: (0, 0)) for w in ws]
    return pl.pallas_call(
        _mlp_body,
        grid=grid,
        in_specs=[
            pl.BlockSpec((_BB, F * K), lambda i: (i, 0)),
            pl.BlockSpec((_BB, F), lambda i: (i, 0)),
            *w_specs,
            pl.BlockSpec((F, H), lambda i: (0, 0)),
            pl.BlockSpec((H, 1), lambda i: (0, 0)),
            pl.BlockSpec(memory_space=pltpu.SMEM),
        ],
        out_specs=pl.BlockSpec((_BB, 1), lambda i: (i, 0)),
        out_shape=jax.ShapeDtypeStruct((B, 1), jnp.float32),
    )(v2, wv, *ws, ball, clfh, scal)


def kernel(x, W0, W_tab, V_tab, mlp_ws, mlp_bs, clf_w, clf_b):
    x_flat = x.reshape(-1).astype(jnp.int32)
    v_flat, w_flat = _make_gather()(x_flat, V_tab, W_tab.reshape(-1))
    v2 = v_flat.reshape(B, F * K)
    wv = w_flat.reshape(B, F)
    ball = jnp.stack(mlp_bs)
    clfh = clf_w[1:]
    scal = jnp.concatenate([W0.reshape(-1), clf_w[0].reshape(-1),
                            clf_b.reshape(-1)])
    return _deepfm_tc(v2, wv, list(mlp_ws), ball, clfh, scal)


# speedup vs baseline: 2.0726x; 1.1342x over previous
import functools

import jax
import jax.numpy as jnp
from jax import lax
from jax.experimental import pallas as pl
from jax.experimental.pallas import tpu as pltpu
from jax.experimental.pallas import tpu_sc as plsc

B = 4096
F = 26
K = 64
H = 256
N_LOOKUPS = B * F

_NC = 2
_NS = 16
_NW = _NC * _NS
_BPW = N_LOOKUPS // _NW
_CHUNK = 832
_NCHUNK = _BPW // _CHUNK


def _gather_body(x_hbm, vtab_hbm, wtab_hbm, vout_hbm, wout_hbm,
                 idx_v, vbuf0, vbuf1, wbuf, gsem0, gsem1, wsem):
    wid = lax.axis_index("s") * _NC + lax.axis_index("c")
    base = wid * _BPW
    pltpu.sync_copy(x_hbm.at[pl.ds(base, _BPW)], idx_v)
    wcp = pltpu.async_copy(wtab_hbm.at[idx_v], wbuf, wsem)
    bufs = (vbuf0, vbuf1)
    sems = (gsem0, gsem1)
    cps = [None, None]
    cps[0] = pltpu.async_copy(
        vtab_hbm.at[idx_v.at[pl.ds(0, _CHUNK)]], bufs[0], sems[0])
    for c in range(_NCHUNK):
        cur = c % 2
        nxt = (c + 1) % 2
        if c + 1 < _NCHUNK:
            cps[nxt] = pltpu.async_copy(
                vtab_hbm.at[idx_v.at[pl.ds((c + 1) * _CHUNK, _CHUNK)]],
                bufs[nxt], sems[nxt])
        cps[cur].wait()
        pltpu.sync_copy(bufs[cur],
                        vout_hbm.at[pl.ds(base + c * _CHUNK, _CHUNK)])
    wcp.wait()
    pltpu.sync_copy(wbuf, wout_hbm.at[pl.ds(base, _BPW)])


@functools.cache
def _make_gather():
    return pl.kernel(
        _gather_body,
        out_type=(
            jax.ShapeDtypeStruct((N_LOOKUPS, K), jnp.float32),
            jax.ShapeDtypeStruct((N_LOOKUPS,), jnp.float32),
        ),
        mesh=plsc.VectorSubcoreMesh(core_axis_name="c",
                                    subcore_axis_name="s"),
        compiler_params=pltpu.CompilerParams(use_tc_tiling_on_sc=False),
        scratch_types=(
            pltpu.VMEM((_BPW,), jnp.int32),
            pltpu.VMEM((_CHUNK, K), jnp.float32),
            pltpu.VMEM((_CHUNK, K), jnp.float32),
            pltpu.VMEM((_BPW,), jnp.float32),
            pltpu.SemaphoreType.DMA,
            pltpu.SemaphoreType.DMA,
            pltpu.SemaphoreType.DMA,
        ),
    )

_BB = 512


def _mlp_body(*refs):
    v_ref, wv_ref = refs[0], refs[1]
    w_refs = refs[2:2 + F]
    ball_ref, clfh_ref, scal_ref = refs[2 + F], refs[3 + F], refs[4 + F]
    out_ref = refs[5 + F]
    v = v_ref[...]
    s2 = v[:, 0:128] * 1.0
    ss2 = v[:, 0:128] * v[:, 0:128]
    for p in range(1, F // 2):
        c = v[:, p * 128:(p + 1) * 128]
        s2 = s2 + c
        ss2 = ss2 + c * c
    s = s2[:, :K] + s2[:, K:]
    ss = ss2[:, :K] + ss2[:, K:]
    fm2 = 0.5 * jnp.sum(s * s - ss, axis=1, keepdims=True)
    fm1 = jnp.sum(wv_ref[...], axis=1, keepdims=True)
    w0 = scal_ref[0]
    clf_w0 = scal_ref[1]
    clf_b = scal_ref[2]
    fm_out = w0 + fm1 + fm2
    h = v
    for i in range(F):
        h = jnp.maximum(
            jnp.dot(h, w_refs[i][...], preferred_element_type=jnp.float32)
            + ball_ref[i, :], 0.0)
    out_ref[...] = (fm_out * clf_w0
                    + jnp.dot(h, clfh_ref[...],
                              preferred_element_type=jnp.float32)
                    + clf_b)


def _deepfm_tc(v2, wv, ws, ball, clfh, scal):
    grid = (B // _BB,)
    w_specs = [pl.BlockSpec(w.shape, lambda i: (0, 0)) for w in ws]
    return pl.pallas_call(
        _mlp_body,
        grid=grid,
        in_specs=[
            pl.BlockSpec((_BB, F * K), lambda i: (i, 0)),
            pl.BlockSpec((_BB, F), lambda i: (i, 0)),
            *w_specs,
            pl.BlockSpec((F, H), lambda i: (0, 0)),
            pl.BlockSpec((H, 1), lambda i: (0, 0)),
            pl.BlockSpec(memory_space=pltpu.SMEM),
        ],
        out_specs=pl.BlockSpec((_BB, 1), lambda i: (i, 0)),
        out_shape=jax.ShapeDtypeStruct((B, 1), jnp.float32),
    )(v2, wv, *ws, ball, clfh, scal)


def kernel(x, W0, W_tab, V_tab, mlp_ws, mlp_bs, clf_w, clf_b):
    x_flat = x.reshape(-1).astype(jnp.int32)
    v_flat, w_flat = _make_gather()(x_flat, V_tab, W_tab.reshape(-1))
    v2 = v_flat.reshape(B, F * K)
    wv = w_flat.reshape(B, F)
    ball = jnp.stack(mlp_bs)
    clfh = clf_w[1:]
    scal = jnp.concatenate([W0.reshape(-1), clf_w[0].reshape(-1),
                            clf_b.reshape(-1)])
    return _deepfm_tc(v2, wv, list(mlp_ws), ball, clfh, scal)


# speedup vs baseline: 2.1870x; 1.0552x over previous
import functools

import jax
import jax.numpy as jnp
from jax import lax
from jax.experimental import pallas as pl
from jax.experimental.pallas import tpu as pltpu
from jax.experimental.pallas import tpu_sc as plsc

B = 4096
F = 26
K = 64
H = 256
N_LOOKUPS = B * F

_NC = 2
_NS = 16
_NW = _NC * _NS
_BPW = N_LOOKUPS // _NW
_CHUNK = 832
_NCHUNK = _BPW // _CHUNK


def _gather_body(x_hbm, vtab_hbm, wtab_hbm, vout_hbm, wout_hbm,
                 idx_v, vbuf0, vbuf1, wbuf, gsem0, gsem1, wsem):
    wid = lax.axis_index("s") * _NC + lax.axis_index("c")
    base = wid * _BPW
    pltpu.sync_copy(x_hbm.at[pl.ds(base, _BPW)], idx_v)
    wcp = pltpu.async_copy(wtab_hbm.at[idx_v], wbuf, wsem)
    bufs = (vbuf0, vbuf1)
    sems = (gsem0, gsem1)
    cps = [None, None]
    cps[0] = pltpu.async_copy(
        vtab_hbm.at[idx_v.at[pl.ds(0, _CHUNK)]], bufs[0], sems[0])
    for c in range(_NCHUNK):
        cur = c % 2
        nxt = (c + 1) % 2
        if c + 1 < _NCHUNK:
            cps[nxt] = pltpu.async_copy(
                vtab_hbm.at[idx_v.at[pl.ds((c + 1) * _CHUNK, _CHUNK)]],
                bufs[nxt], sems[nxt])
        cps[cur].wait()
        pltpu.sync_copy(bufs[cur],
                        vout_hbm.at[pl.ds(base + c * _CHUNK, _CHUNK)])
    wcp.wait()
    pltpu.sync_copy(wbuf, wout_hbm.at[pl.ds(base, _BPW)])


@functools.cache
def _make_gather():
    return pl.kernel(
        _gather_body,
        out_type=(
            jax.ShapeDtypeStruct((N_LOOKUPS, K), jnp.float32),
            jax.ShapeDtypeStruct((N_LOOKUPS,), jnp.float32),
        ),
        mesh=plsc.VectorSubcoreMesh(core_axis_name="c",
                                    subcore_axis_name="s"),
        compiler_params=pltpu.CompilerParams(use_tc_tiling_on_sc=False),
        scratch_types=(
            pltpu.VMEM((_BPW,), jnp.int32),
            pltpu.VMEM((_CHUNK, K), jnp.float32),
            pltpu.VMEM((_CHUNK, K), jnp.float32),
            pltpu.VMEM((_BPW,), jnp.float32),
            pltpu.SemaphoreType.DMA,
            pltpu.SemaphoreType.DMA,
            pltpu.SemaphoreType.DMA,
        ),
    )

_BB = 1024


def _mlp_body(*refs):
    v_ref, wv_ref = refs[0], refs[1]
    w_refs = refs[2:2 + F]
    ball_ref, clfh_ref, scal_ref = refs[2 + F], refs[3 + F], refs[4 + F]
    out_ref = refs[5 + F]
    v = v_ref[...]
    s2 = v[:, 0:128] * 1.0
    ss2 = v[:, 0:128] * v[:, 0:128]
    for p in range(1, F // 2):
        c = v[:, p * 128:(p + 1) * 128]
        s2 = s2 + c
        ss2 = ss2 + c * c
    s = s2[:, :K] + s2[:, K:]
    ss = ss2[:, :K] + ss2[:, K:]
    fm2 = 0.5 * jnp.sum(s * s - ss, axis=1, keepdims=True)
    fm1 = jnp.sum(wv_ref[...], axis=1, keepdims=True)
    w0 = scal_ref[0]
    clf_w0 = scal_ref[1]
    clf_b = scal_ref[2]
    fm_out = w0 + fm1 + fm2
    h = v
    for i in range(F):
        h = jnp.maximum(
            jnp.dot(h, w_refs[i][...], preferred_element_type=jnp.float32)
            + ball_ref[i, :], 0.0)
    out_ref[...] = (fm_out * clf_w0
                    + jnp.dot(h, clfh_ref[...],
                              preferred_element_type=jnp.float32)
                    + clf_b)


def _deepfm_tc(v2, wv, ws, ball, clfh, scal):
    grid = (B // _BB,)
    w_specs = [pl.BlockSpec(w.shape, lambda i: (0, 0)) for w in ws]
    return pl.pallas_call(
        _mlp_body,
        grid=grid,
        in_specs=[
            pl.BlockSpec((_BB, F * K), lambda i: (i, 0)),
            pl.BlockSpec((_BB, F), lambda i: (i, 0)),
            *w_specs,
            pl.BlockSpec((F, H), lambda i: (0, 0)),
            pl.BlockSpec((H, 1), lambda i: (0, 0)),
            pl.BlockSpec(memory_space=pltpu.SMEM),
        ],
        out_specs=pl.BlockSpec((_BB, 1), lambda i: (i, 0)),
        out_shape=jax.ShapeDtypeStruct((B, 1), jnp.float32),
    )(v2, wv, *ws, ball, clfh, scal)


def kernel(x, W0, W_tab, V_tab, mlp_ws, mlp_bs, clf_w, clf_b):
    x_flat = x.reshape(-1).astype(jnp.int32)
    v_flat, w_flat = _make_gather()(x_flat, V_tab, W_tab.reshape(-1))
    v2 = v_flat.reshape(B, F * K)
    wv = w_flat.reshape(B, F)
    ball = jnp.stack(mlp_bs)
    clfh = clf_w[1:]
    scal = jnp.concatenate([W0.reshape(-1), clf_w[0].reshape(-1),
                            clf_b.reshape(-1)])
    return _deepfm_tc(v2, wv, list(mlp_ws), ball, clfh, scal)
